# Initial kernel scaffold; baseline (speedup 1.0000x reference)
import jax
import jax.numpy as jnp
from jax.experimental import pallas as pl


def kernel(x, edge_index, W1l, b1l, W1r, W2l, b2l, W2r, W3l, b3l, W3r, W4l, b4l, W4r):
    raise NotImplementedError("write your pallas kernel here")


# speedup vs baseline: 5.3998x; 5.3998x over previous
import jax
import jax.numpy as jnp
from jax import lax
from jax.experimental import pallas as pl
from jax.experimental.pallas import tpu as pltpu
from jax.experimental.pallas import tpu_sc as plsc

N = 10000
E = 320000
D = 128
H = 64

NC = 2
NS = 16
NT = NC * NS
ROWS_PER_TILE = 640
PAD_N = NS * ROWS_PER_TILE
CHUNK = 128
CH_PER_STEP = 8
CHUNKS_PER_TILE = 80
STEPS = CHUNKS_PER_TILE // CH_PER_STEP
E_PAD = NT * CHUNKS_PER_TILE * CHUNK


def _make_sc_agg(with_deg):
  out_types = [jax.ShapeDtypeStruct((NC, PAD_N, H), jnp.float32)]
  if with_deg:
    out_types.append(jax.ShapeDtypeStruct((NC, PAD_N, 16), jnp.float32))
  scratch = [
      pltpu.VMEM((CH_PER_STEP, CHUNK), jnp.int32),
      pltpu.VMEM((CH_PER_STEP, CHUNK), jnp.int32),
      pltpu.VMEM((CH_PER_STEP, CHUNK, H), jnp.float32),
      pltpu.VMEM_SHARED((PAD_N, H), jnp.float32),
      pltpu.SemaphoreType.DMA,
  ]
  if with_deg:
    scratch += [
        pltpu.VMEM((CHUNK, 16), jnp.float32),
        pltpu.VMEM_SHARED((PAD_N, 16), jnp.float32),
    ]

  def body(y_hbm, src_hbm, dst_hbm, z64_hbm, *rest):
    if with_deg:
      (z16_hbm, ones_hbm, out_hbm, dout_hbm,
       srcv, dstv, rows, acc, sem, onesv, dacc) = rest
    else:
      (out_hbm, srcv, dstv, rows, acc, sem) = rest
    cid = lax.axis_index("c")
    sid = lax.axis_index("s")
    wid = cid * NS + sid
    r0 = sid * ROWS_PER_TILE
    pltpu.sync_copy(z64_hbm.at[pl.ds(r0, ROWS_PER_TILE)],
                    acc.at[pl.ds(r0, ROWS_PER_TILE)])
    if with_deg:
      pltpu.sync_copy(z16_hbm.at[pl.ds(r0, ROWS_PER_TILE)],
                      dacc.at[pl.ds(r0, ROWS_PER_TILE)])
      pltpu.sync_copy(ones_hbm, onesv)
    plsc.subcore_barrier()

    def step(i, carry):
      base = wid * CHUNKS_PER_TILE + i * CH_PER_STEP
      pltpu.sync_copy(src_hbm.at[pl.ds(base, CH_PER_STEP)], srcv)
      pltpu.sync_copy(dst_hbm.at[pl.ds(base, CH_PER_STEP)], dstv)
      cps = [pltpu.async_copy(y_hbm.at[srcv.at[j]], rows.at[j], sem)
             for j in range(CH_PER_STEP)]
      for c in cps:
        c.wait()
      for j in range(CH_PER_STEP):
        pltpu.sync_copy(rows.at[j], acc.at[dstv.at[j]], add=True)
        if with_deg:
          pltpu.sync_copy(onesv, dacc.at[dstv.at[j]], add=True)
      return carry

    lax.fori_loop(0, STEPS, step, 0)
    plsc.subcore_barrier()
    pltpu.sync_copy(acc.at[pl.ds(r0, ROWS_PER_TILE)],
                    out_hbm.at[cid, pl.ds(r0, ROWS_PER_TILE)])
    if with_deg:
      pltpu.sync_copy(dacc.at[pl.ds(r0, ROWS_PER_TILE)],
                      dout_hbm.at[cid, pl.ds(r0, ROWS_PER_TILE)])

  mesh = plsc.VectorSubcoreMesh(core_axis_name="c", subcore_axis_name="s")
  return pl.kernel(
      body, out_type=tuple(out_types), mesh=mesh, scratch_types=scratch,
      compiler_params=pltpu.CompilerParams(use_tc_tiling_on_sc=False))


import functools


@functools.lru_cache(maxsize=None)
def _get_sc_kernel(with_deg):
  return _make_sc_agg(with_deg)


def _sc_agg_deg(*args):
  return _get_sc_kernel(True)(*args)


def _sc_agg(*args):
  return _get_sc_kernel(False)(*args)


_BR = 2000


def _dot_t(a, w):
  return lax.dot_general(a, w, (((1,), (1,)), ((), ())),
                         preferred_element_type=jnp.float32)


def _pre_kernel(x_ref, wl_ref, wr_ref, y_ref, r_ref):
  xb = x_ref[...]
  y_ref[...] = _dot_t(xb, wl_ref[...])
  r_ref[...] = _dot_t(xb, wr_ref[...])


def _tc_pre(x, wl, wr):
  n, d = x.shape
  h = wl.shape[0]
  return pl.pallas_call(
      _pre_kernel,
      grid=(n // _BR,),
      in_specs=[pl.BlockSpec((_BR, d), lambda i: (i, 0)),
                pl.BlockSpec((h, d), lambda i: (0, 0)),
                pl.BlockSpec((h, d), lambda i: (0, 0))],
      out_specs=[pl.BlockSpec((_BR, h), lambda i: (i, 0)),
                 pl.BlockSpec((_BR, h), lambda i: (i, 0))],
      out_shape=[jax.ShapeDtypeStruct((n, h), jnp.float32),
                 jax.ShapeDtypeStruct((n, h), jnp.float32)],
  )(x, wl, wr)


def _mean_term(p0_ref, p1_ref, d0_ref, d1_ref):
  deg = d0_ref[0][:, :1] + d1_ref[0][:, :1]
  return (p0_ref[0] + p1_ref[0]) / jnp.maximum(deg, 1.0)


def _make_mid_kernel(with_res):
  def kern(p0, p1, d0, d1, b, rc, *rest):
    if with_res:
      res, wl, wr, ho, yo, ro = rest
    else:
      wl, wr, ho, yo, ro = rest
    m = _mean_term(p0, p1, d0, d1) + b[...] + rc[...]
    hh = jnp.maximum(m, 0.0)
    if with_res:
      hh = hh + res[...]
    ho[...] = hh
    yo[...] = _dot_t(hh, wl[...])
    ro[...] = _dot_t(hh, wr[...])
  return kern


def _tc_mid(p, dp, b, rc, res, wl, wr):
  with_res = res is not None
  in_specs = [
      pl.BlockSpec((1, _BR, H), lambda i: (0, i, 0)),
      pl.BlockSpec((1, _BR, H), lambda i: (1, i, 0)),
      pl.BlockSpec((1, _BR, 16), lambda i: (0, i, 0)),
      pl.BlockSpec((1, _BR, 16), lambda i: (1, i, 0)),
      pl.BlockSpec((1, H), lambda i: (0, 0)),
      pl.BlockSpec((_BR, H), lambda i: (i, 0)),
  ]
  args = [p, p, dp, dp, b, rc]
  if with_res:
    in_specs.append(pl.BlockSpec((_BR, H), lambda i: (i, 0)))
    args.append(res)
  in_specs += [pl.BlockSpec((H, H), lambda i: (0, 0)),
               pl.BlockSpec((H, H), lambda i: (0, 0))]
  args += [wl, wr]
  return pl.pallas_call(
      _make_mid_kernel(with_res),
      grid=(N // _BR,),
      in_specs=in_specs,
      out_specs=[pl.BlockSpec((_BR, H), lambda i: (i, 0))] * 3,
      out_shape=[jax.ShapeDtypeStruct((N, H), jnp.float32)] * 3,
  )(*args)


def _final_kernel(p0, p1, d0, d1, b, rc, out):
  o = _mean_term(p0, p1, d0, d1) + b[...] + rc[...]
  nrm = jnp.sqrt(jnp.sum(o * o, axis=1, keepdims=True))
  out[...] = o / jnp.maximum(nrm, 1e-12)


def _tc_final(p, dp, b, rc):
  return pl.pallas_call(
      _final_kernel,
      grid=(N // _BR,),
      in_specs=[
          pl.BlockSpec((1, _BR, H), lambda i: (0, i, 0)),
          pl.BlockSpec((1, _BR, H), lambda i: (1, i, 0)),
          pl.BlockSpec((1, _BR, 16), lambda i: (0, i, 0)),
          pl.BlockSpec((1, _BR, 16), lambda i: (1, i, 0)),
          pl.BlockSpec((1, H), lambda i: (0, 0)),
          pl.BlockSpec((_BR, H), lambda i: (i, 0)),
      ],
      out_specs=pl.BlockSpec((_BR, H), lambda i: (i, 0)),
      out_shape=jax.ShapeDtypeStruct((N, H), jnp.float32),
  )(p, p, dp, dp, b, rc)


def kernel(x, edge_index, W1l, b1l, W1r, W2l, b2l, W2r,
           W3l, b3l, W3r, W4l, b4l, W4r):
  src = edge_index[0]
  dst = edge_index[1]
  pad = E_PAD - E
  src2 = jnp.concatenate(
      [src, jnp.zeros((pad,), jnp.int32)]).reshape(E_PAD // CHUNK, CHUNK)
  dst2 = jnp.concatenate(
      [dst, jnp.full((pad,), N, jnp.int32)]).reshape(E_PAD // CHUNK, CHUNK)
  z64 = jnp.zeros((PAD_N, H), jnp.float32)
  z16 = jnp.zeros((PAD_N, 16), jnp.float32)
  ones16 = jnp.ones((CHUNK, 16), jnp.float32)
  b1 = b1l.reshape(1, H)
  b2 = b2l.reshape(1, H)
  b3 = b3l.reshape(1, H)
  b4 = b4l.reshape(1, H)

  y1, r1 = _tc_pre(x, W1l, W1r)
  p1, dp = _sc_agg_deg(y1, src2, dst2, z64, z16, ones16)
  h1, y2, r2 = _tc_mid(p1, dp, b1, r1, None, W2l, W2r)
  (p2,) = _sc_agg(y2, src2, dst2, z64)
  h2, y3, r3 = _tc_mid(p2, dp, b2, r2, h1, W3l, W3r)
  (p3,) = _sc_agg(y3, src2, dst2, z64)
  h3, y4, r4 = _tc_mid(p3, dp, b3, r3, h2, W4l, W4r)
  (p4,) = _sc_agg(y4, src2, dst2, z64)
  return _tc_final(p4, dp, b4, r4)
